# Initial kernel scaffold; baseline (speedup 1.0000x reference)
import jax
import jax.numpy as jnp
from jax.experimental import pallas as pl


def kernel(qw1_location, qw1_log_scale, qw0_location, qw0_log_scale, qz2_location, qz2_log_scale, qz1_location, qz1_log_scale, datapoints_indices, counts, holdout_mask, sample_key):
    raise NotImplementedError("write your pallas kernel here")


# speedup vs baseline: 1.6406x; 1.6406x over previous
import functools
import math

import numpy as np
import jax
import jax.numpy as jnp
from jax.scipy.special import gammaln
from jax.experimental import pallas as pl
from jax.experimental.pallas import tpu as pltpu

_LOG_2PI = math.log(2.0 * math.pi)
_LANES = 128
_CONC = 0.1

_N = 4096
_D = 4096
_L0 = 256
_L1 = 128
_S = 8
_B = 1024

_GROUPS = (
    ("qw1", (_L1, _L0), 0.3, 128),
    ("qw0", (_L0, _D), 0.3, 512),
    ("qz2", (_N, _L1), 0.3, 512),
    ("qz1", (_N, _L0), 2.7, 512),
)

_LO = np.float32(np.nextafter(np.float32(-1.0), np.float32(0.0)))
_DIFF = np.float32(np.float32(1.0) - _LO)
_SQRT2 = np.float32(np.sqrt(2.0))


def _rotl(x, r):
    return (x << np.uint32(r)) | jax.lax.shift_right_logical(x, np.uint32(32 - r))


def _threefry2x32(k0, k1, x0, x1):
    ks2 = k0 ^ k1 ^ np.uint32(0x1BD11BDA)
    x0 = x0 + k0
    x1 = x1 + k1
    rots = ((13, 15, 26, 6), (17, 29, 16, 24))
    sched = ((k1, ks2, 1), (ks2, k0, 2), (k0, k1, 3), (k1, ks2, 4), (ks2, k0, 5))
    for i, (a, b, c) in enumerate(sched):
        for r in rots[i % 2]:
            x0 = x0 + x1
            x1 = _rotl(x1, r)
            x1 = x1 ^ x0
        x0 = x0 + a
        x1 = x1 + b + np.uint32(c)
    return x0, x1


def _erfinv(x):
    w = -jnp.log1p(-x * x)
    ws = w - 2.5
    wb = jnp.sqrt(w) - 3.0
    p_s = jnp.float32(2.81022636e-08)
    for c in (3.43273939e-07, -3.5233877e-06, -4.39150654e-06, 0.00021858087,
              -0.00125372503, -0.00417768164, 0.246640727, 1.50140941):
        p_s = p_s * ws + np.float32(c)
    p_b = jnp.float32(-0.000200214257)
    for c in (0.000100950558, 0.00134934322, -0.00367342844, 0.00573950773,
              -0.0076224613, 0.00943887047, 1.00167406, 2.83297682):
        p_b = p_b * wb + np.float32(c)
    return jnp.where(w < 5.0, p_s, p_b) * x


def _sampler_kernel(key_ref, loc_ref, ls_ref, x_ref, sums_ref, scale_scr,
                    *, rows_total, tile_r, rate):
    r = pl.program_id(0)
    sh = pl.program_id(1)

    @pl.when(sh == 0)
    def _():
        ls = ls_ref[...]
        sp = jnp.maximum(ls, 0.0) + jnp.log1p(jnp.exp(-jnp.abs(ls)))
        scale = jnp.maximum(sp, 1e-5)
        scale_scr[...] = scale
        sums_ref[0, 0, 0] = 0.0
        sums_ref[0, 0, 1] = 0.0
        sums_ref[0, 0, 2] = jnp.sum(jnp.log(scale))
        sums_ref[0, 0, 3] = 0.0

    k0 = key_ref[0]
    k1 = key_ref[1]
    loc = loc_ref[...]
    scale = scale_scr[...]

    i = jax.lax.broadcasted_iota(jnp.int32, (tile_r, _LANES), 0)
    j = jax.lax.broadcasted_iota(jnp.int32, (tile_r, _LANES), 1)
    vec = i * _LANES + j

    def half(s_idx):
        off = (s_idx * rows_total + r * tile_r) * _LANES
        cnt = (vec + off).astype(jnp.uint32)
        b0, b1 = _threefry2x32(k0, k1, jnp.zeros_like(cnt), cnt)
        bits = b0 ^ b1
        fb = pltpu.bitcast((bits >> np.uint32(9)) | np.uint32(0x3F800000),
                           jnp.float32) - 1.0
        u = fb * _DIFF + _LO
        eps = _SQRT2 * _erfinv(u)
        y = loc + scale * eps
        xv = jnp.exp(y)
        x_ref[pl.ds(s_idx, 1)] = xv[None]
        lp = jnp.sum((_CONC - 1.0) * y - rate * xv)
        ent = jnp.sum(0.5 * (eps * eps) + y)
        return lp, ent

    lp0, ent0 = half(sh)
    lp1, ent1 = half(sh + _S // 2)
    sums_ref[0, 0, 0] += lp0 + lp1
    sums_ref[0, 0, 1] += ent0 + ent1


def _sample_group(kd, loc, log_scale, rate, tile_r):
    n0, n1 = loc.shape
    rows = (n0 * n1) // _LANES
    rt = rows // tile_r
    loc2 = loc.reshape(rows, _LANES)
    ls2 = log_scale.reshape(rows, _LANES)
    x, sums = pl.pallas_call(
        functools.partial(_sampler_kernel, rows_total=rows, tile_r=tile_r,
                          rate=np.float32(rate)),
        out_shape=(jax.ShapeDtypeStruct((_S, rows, _LANES), jnp.float32),
                   jax.ShapeDtypeStruct((rt, 1, 4), jnp.float32)),
        grid_spec=pltpu.PrefetchScalarGridSpec(
            num_scalar_prefetch=0,
            grid=(rt, _S // 2),
            in_specs=[
                pl.BlockSpec(memory_space=pltpu.MemorySpace.SMEM),
                pl.BlockSpec((tile_r, _LANES), lambda r, s: (r, 0)),
                pl.BlockSpec((tile_r, _LANES), lambda r, s: (r, 0)),
            ],
            out_specs=[
                pl.BlockSpec((_S, tile_r, _LANES), lambda r, s: (0, r, 0)),
                pl.BlockSpec((1, 1, 4), lambda r, s: (r, 0, 0),
                             memory_space=pltpu.MemorySpace.SMEM),
            ],
            scratch_shapes=[pltpu.VMEM((tile_r, _LANES), jnp.float32)]),
        compiler_params=pltpu.CompilerParams(
            dimension_semantics=("parallel", "arbitrary"),
            vmem_limit_bytes=48 * 1024 * 1024),
    )(kd, loc2, ls2)
    return x.reshape(_S, n0, n1), sums


def _ll_kernel(z1_ref, w0_ref, counts_ref, mask_ref, out_ref):
    counts = counts_ref[...]
    keep = 1.0 - mask_ref[...]
    ck = counts * keep
    rate = jnp.dot(z1_ref[0], w0_ref[0].astype(jnp.bfloat16),
                   preferred_element_type=jnp.float32)
    prod = rate
    ssum = rate
    for s in range(1, _S):
        rate = jnp.dot(z1_ref[s], w0_ref[s].astype(jnp.bfloat16),
                       preferred_element_type=jnp.float32)
        prod = prod * rate
        ssum = ssum + rate
    log_prod = jnp.log(jnp.clip(prod, 1e-30, 1e30))
    out_ref[0, 0, 0] = jnp.sum(ck * log_prod - keep * ssum)


def _data_ll_sums(z1_bf16, w0, counts, mask, tile_d):
    dt = _D // tile_d
    return pl.pallas_call(
        _ll_kernel,
        out_shape=jax.ShapeDtypeStruct((dt, 1, 1), jnp.float32),
        grid_spec=pltpu.PrefetchScalarGridSpec(
            num_scalar_prefetch=0,
            grid=(dt,),
            in_specs=[
                pl.BlockSpec((_S, _B, _L0), lambda d: (0, 0, 0)),
                pl.BlockSpec((_S, _L0, tile_d), lambda d: (0, 0, d)),
                pl.BlockSpec((_B, tile_d), lambda d: (0, d)),
                pl.BlockSpec((_B, tile_d), lambda d: (0, d)),
            ],
            out_specs=pl.BlockSpec((1, 1, 1), lambda d: (d, 0, 0),
                                   memory_space=pltpu.MemorySpace.SMEM)),
        compiler_params=pltpu.CompilerParams(
            dimension_semantics=("parallel",),
            vmem_limit_bytes=48 * 1024 * 1024),
    )(z1_bf16, w0, counts, mask)


def kernel(qw1_location, qw1_log_scale, qw0_location, qw0_log_scale,
           qz2_location, qz2_log_scale, qz1_location, qz1_log_scale,
           datapoints_indices, counts, holdout_mask, sample_key):
    params = {
        "qw1": (qw1_location, qw1_log_scale),
        "qw0": (qw0_location, qw0_log_scale),
        "qz2": (qz2_location, qz2_log_scale),
        "qz1": (qz1_location, qz1_log_scale),
    }
    key = jax.random.wrap_key_data(sample_key)
    keys = jax.random.split(key, len(_GROUPS))
    kds = jax.vmap(jax.random.key_data)(keys)

    samples = {}
    lp_sum = 0.0
    ent_sum = 0.0
    logscale_sum = 0.0
    lp_const = 0.0
    total = 0
    for gi, (name, shape, rate, tile_r) in enumerate(_GROUPS):
        loc, ls = params[name]
        x, sums = _sample_group(kds[gi], loc, ls, rate, tile_r)
        samples[name] = x
        lp_sum = lp_sum + jnp.sum(sums[:, 0, 0])
        ent_sum = ent_sum + jnp.sum(sums[:, 0, 1])
        logscale_sum = logscale_sum + jnp.sum(sums[:, 0, 2])
        n = shape[0] * shape[1]
        total += n
        lp_const += n * (_CONC * math.log(rate) - math.lgamma(_CONC))

    log_prior = lp_sum / _S + lp_const
    entropy = ent_sum / _S + logscale_sum + total * (0.5 * _LOG_2PI)

    z1_bf16 = jnp.take(samples["qz1"], datapoints_indices,
                       axis=1).astype(jnp.bfloat16)
    counts_f = counts.astype(jnp.float32)
    parts = _data_ll_sums(z1_bf16, samples["qw0"], counts_f, holdout_mask,
                          tile_d=512)
    ll_sum = jnp.sum(parts[:, 0, 0])
    keep = 1.0 - holdout_mask.astype(jnp.float32)
    lgc_keep = jnp.sum(keep * gammaln(counts_f + 1.0))
    data_ll = (_N / _B) * (ll_sum / (_S * _D) - lgc_keep / _D)

    elbo = data_ll + log_prior + entropy
    return elbo, (log_prior, entropy, data_ll), samples


# speedup vs baseline: 1.7149x; 1.0453x over previous
import functools
import math

import numpy as np
import jax
import jax.numpy as jnp
from jax.experimental import pallas as pl
from jax.experimental.pallas import tpu as pltpu

_LOG_2PI = math.log(2.0 * math.pi)
_LANES = 128
_CONC = 0.1

_N = 4096
_D = 4096
_L0 = 256
_L1 = 128
_S = 8
_B = 1024

_GROUPS = (
    ("qw1", (_L1, _L0), 0.3, 128),
    ("qw0", (_L0, _D), 0.3, 512),
    ("qz2", (_N, _L1), 0.3, 512),
    ("qz1", (_N, _L0), 2.7, 512),
)

_LO = np.float32(np.nextafter(np.float32(-1.0), np.float32(0.0)))
_DIFF = np.float32(np.float32(1.0) - _LO)
_SQRT2 = np.float32(np.sqrt(2.0))


def _rotl(x, r):
    return (x << np.uint32(r)) | jax.lax.shift_right_logical(x, np.uint32(32 - r))


def _threefry2x32(k0, k1, x0, x1):
    ks2 = k0 ^ k1 ^ np.uint32(0x1BD11BDA)
    x0 = x0 + k0
    x1 = x1 + k1
    rots = ((13, 15, 26, 6), (17, 29, 16, 24))
    sched = ((k1, ks2, 1), (ks2, k0, 2), (k0, k1, 3), (k1, ks2, 4), (ks2, k0, 5))
    for i, (a, b, c) in enumerate(sched):
        for r in rots[i % 2]:
            x0 = x0 + x1
            x1 = _rotl(x1, r)
            x1 = x1 ^ x0
        x0 = x0 + a
        x1 = x1 + b + np.uint32(c)
    return x0, x1


def _erfinv(x):
    w = -jnp.log1p(-x * x)
    ws = w - 2.5
    wb = jnp.sqrt(w) - 3.0
    p_s = jnp.float32(2.81022636e-08)
    for c in (3.43273939e-07, -3.5233877e-06, -4.39150654e-06, 0.00021858087,
              -0.00125372503, -0.00417768164, 0.246640727, 1.50140941):
        p_s = p_s * ws + np.float32(c)
    p_b = jnp.float32(-0.000200214257)
    for c in (0.000100950558, 0.00134934322, -0.00367342844, 0.00573950773,
              -0.0076224613, 0.00943887047, 1.00167406, 2.83297682):
        p_b = p_b * wb + np.float32(c)
    return jnp.where(w < 5.0, p_s, p_b) * x


def _sampler_kernel(key_ref, loc_ref, ls_ref, *refs,
                    rows_total, tile_r, rate, emit_bf16):
    if emit_bf16:
        x_ref, xb_ref, sums_ref, scale_scr = refs
    else:
        x_ref, sums_ref, scale_scr = refs
        xb_ref = None
    r = pl.program_id(0)
    sh = pl.program_id(1)

    @pl.when(sh == 0)
    def _():
        ls = ls_ref[...]
        sp = jnp.maximum(ls, 0.0) + jnp.log1p(jnp.exp(-jnp.abs(ls)))
        scale = jnp.maximum(sp, 1e-5)
        scale_scr[...] = scale
        sums_ref[0, 0, 0] = 0.0
        sums_ref[0, 0, 1] = 0.0
        sums_ref[0, 0, 2] = jnp.sum(jnp.log(scale))
        sums_ref[0, 0, 3] = 0.0

    k0 = key_ref[0]
    k1 = key_ref[1]
    loc = loc_ref[...]
    scale = scale_scr[...]

    i = jax.lax.broadcasted_iota(jnp.int32, (tile_r, _LANES), 0)
    j = jax.lax.broadcasted_iota(jnp.int32, (tile_r, _LANES), 1)
    vec = i * _LANES + j

    def half(s_idx):
        off = (s_idx * rows_total + r * tile_r) * _LANES
        cnt = (vec + off).astype(jnp.uint32)
        b0, b1 = _threefry2x32(k0, k1, jnp.zeros_like(cnt), cnt)
        bits = b0 ^ b1
        fb = pltpu.bitcast((bits >> np.uint32(9)) | np.uint32(0x3F800000),
                           jnp.float32) - 1.0
        u = fb * _DIFF + _LO
        eps = _SQRT2 * _erfinv(u)
        y = loc + scale * eps
        xv = jnp.exp(y)
        x_ref[pl.ds(s_idx, 1)] = xv[None]
        if emit_bf16:
            xb_ref[pl.ds(s_idx, 1)] = xv.astype(jnp.bfloat16)[None]
        lp = jnp.sum((_CONC - 1.0) * y - rate * xv)
        ent = jnp.sum(0.5 * (eps * eps) + y)
        return lp, ent

    lp0, ent0 = half(sh)
    lp1, ent1 = half(sh + _S // 2)
    sums_ref[0, 0, 0] += lp0 + lp1
    sums_ref[0, 0, 1] += ent0 + ent1


def _sample_group(kd, loc, log_scale, rate, tile_r, emit_bf16=False):
    n0, n1 = loc.shape
    rows = (n0 * n1) // _LANES
    rt = rows // tile_r
    loc2 = loc.reshape(rows, _LANES)
    ls2 = log_scale.reshape(rows, _LANES)
    out_shape = [jax.ShapeDtypeStruct((_S, rows, _LANES), jnp.float32)]
    out_specs = [pl.BlockSpec((_S, tile_r, _LANES), lambda r, s: (0, r, 0))]
    if emit_bf16:
        out_shape.append(jax.ShapeDtypeStruct((_S, rows, _LANES), jnp.bfloat16))
        out_specs.append(pl.BlockSpec((_S, tile_r, _LANES),
                                      lambda r, s: (0, r, 0)))
    out_shape.append(jax.ShapeDtypeStruct((rt, 1, 4), jnp.float32))
    out_specs.append(pl.BlockSpec((1, 1, 4), lambda r, s: (r, 0, 0),
                                  memory_space=pltpu.MemorySpace.SMEM))
    outs = pl.pallas_call(
        functools.partial(_sampler_kernel, rows_total=rows, tile_r=tile_r,
                          rate=np.float32(rate), emit_bf16=emit_bf16),
        out_shape=tuple(out_shape),
        grid_spec=pltpu.PrefetchScalarGridSpec(
            num_scalar_prefetch=0,
            grid=(rt, _S // 2),
            in_specs=[
                pl.BlockSpec(memory_space=pltpu.MemorySpace.SMEM),
                pl.BlockSpec((tile_r, _LANES), lambda r, s: (r, 0)),
                pl.BlockSpec((tile_r, _LANES), lambda r, s: (r, 0)),
            ],
            out_specs=out_specs,
            scratch_shapes=[pltpu.VMEM((tile_r, _LANES), jnp.float32)]),
        compiler_params=pltpu.CompilerParams(
            dimension_semantics=("parallel", "arbitrary"),
            vmem_limit_bytes=48 * 1024 * 1024),
    )(kd, loc2, ls2)
    if emit_bf16:
        x, xb, sums = outs
        return x.reshape(_S, n0, n1), xb.reshape(_S, n0, n1), sums
    x, sums = outs
    return x.reshape(_S, n0, n1), None, sums


def _lgamma_p1(c):
    x = c + 9.0
    p = c + 1.0
    for k in range(2, 9):
        p = p * (c + k)
    lz = jnp.log(x)
    return ((x - 0.5) * lz - x + np.float32(0.5 * _LOG_2PI)
            + 1.0 / (12.0 * x) - 1.0 / (360.0 * x * x * x) - jnp.log(p))


def _ll_kernel(z1_ref, w0_ref, counts_ref, mask_ref, out_ref):
    counts = counts_ref[...]
    keep = 1.0 - mask_ref[...]
    ck = counts * keep
    rate = jnp.dot(z1_ref[0], w0_ref[0], preferred_element_type=jnp.float32)
    prod = rate
    ssum = rate
    for s in range(1, _S):
        rate = jnp.dot(z1_ref[s], w0_ref[s], preferred_element_type=jnp.float32)
        prod = prod * rate
        ssum = ssum + rate
    log_prod = jnp.log(jnp.clip(prod, 1e-30, 1e30))
    out_ref[0, 0, 0] = jnp.sum(ck * log_prod - keep * ssum)
    out_ref[0, 0, 1] = jnp.sum(keep * _lgamma_p1(counts))


def _data_ll_sums(z1_bf16, w0, counts, mask, tile_d):
    dt = _D // tile_d
    return pl.pallas_call(
        _ll_kernel,
        out_shape=jax.ShapeDtypeStruct((dt, 1, 2), jnp.float32),
        grid_spec=pltpu.PrefetchScalarGridSpec(
            num_scalar_prefetch=0,
            grid=(dt,),
            in_specs=[
                pl.BlockSpec((_S, _B, _L0), lambda d: (0, 0, 0)),
                pl.BlockSpec((_S, _L0, tile_d), lambda d: (0, 0, d)),
                pl.BlockSpec((_B, tile_d), lambda d: (0, d)),
                pl.BlockSpec((_B, tile_d), lambda d: (0, d)),
            ],
            out_specs=pl.BlockSpec((1, 1, 2), lambda d: (d, 0, 0),
                                   memory_space=pltpu.MemorySpace.SMEM)),
        compiler_params=pltpu.CompilerParams(
            dimension_semantics=("parallel",),
            vmem_limit_bytes=48 * 1024 * 1024),
    )(z1_bf16, w0, counts, mask)


def kernel(qw1_location, qw1_log_scale, qw0_location, qw0_log_scale,
           qz2_location, qz2_log_scale, qz1_location, qz1_log_scale,
           datapoints_indices, counts, holdout_mask, sample_key):
    params = {
        "qw1": (qw1_location, qw1_log_scale),
        "qw0": (qw0_location, qw0_log_scale),
        "qz2": (qz2_location, qz2_log_scale),
        "qz1": (qz1_location, qz1_log_scale),
    }
    key = jax.random.wrap_key_data(sample_key)
    keys = jax.random.split(key, len(_GROUPS))
    kds = jax.vmap(jax.random.key_data)(keys)

    samples = {}
    samples_bf16 = {}
    lp_sum = 0.0
    ent_sum = 0.0
    logscale_sum = 0.0
    lp_const = 0.0
    total = 0
    for gi, (name, shape, rate, tile_r) in enumerate(_GROUPS):
        loc, ls = params[name]
        x, xb, sums = _sample_group(kds[gi], loc, ls, rate, tile_r,
                                    emit_bf16=name in ("qw0", "qz1"))
        samples[name] = x
        samples_bf16[name] = xb
        lp_sum = lp_sum + jnp.sum(sums[:, 0, 0])
        ent_sum = ent_sum + jnp.sum(sums[:, 0, 1])
        logscale_sum = logscale_sum + jnp.sum(sums[:, 0, 2])
        n = shape[0] * shape[1]
        total += n
        lp_const += n * (_CONC * math.log(rate) - math.lgamma(_CONC))

    log_prior = lp_sum / _S + lp_const
    entropy = ent_sum / _S + logscale_sum + total * (0.5 * _LOG_2PI)

    z1_bf16 = jnp.take(samples_bf16["qz1"], datapoints_indices, axis=1)
    counts_f = counts.astype(jnp.float32)
    parts = _data_ll_sums(z1_bf16, samples_bf16["qw0"], counts_f, holdout_mask,
                          tile_d=512)
    ll_sum = jnp.sum(parts[:, 0, 0])
    lgc_keep = jnp.sum(parts[:, 0, 1])
    data_ll = (_N / _B) * (ll_sum / (_S * _D) - lgc_keep / _D)

    elbo = data_ll + log_prior + entropy
    return elbo, (log_prior, entropy, data_ll), samples


# speedup vs baseline: 2.1792x; 1.2708x over previous
import functools
import math

import numpy as np
import jax
import jax.numpy as jnp
from jax.experimental import pallas as pl
from jax.experimental.pallas import tpu as pltpu

_LOG_2PI = math.log(2.0 * math.pi)
_LANES = 128
_CONC = 0.1

_N = 4096
_D = 4096
_L0 = 256
_L1 = 128
_S = 8
_B = 1024

_GROUPS = (
    ("qw1", (_L1, _L0), 0.3, 128),
    ("qw0", (_L0, _D), 0.3, 512),
    ("qz2", (_N, _L1), 0.3, 512),
    ("qz1", (_N, _L0), 2.7, 512),
)

_LO = np.float32(np.nextafter(np.float32(-1.0), np.float32(0.0)))
_DIFF = np.float32(np.float32(1.0) - _LO)
_SQRT2 = np.float32(np.sqrt(2.0))
_CHUNK_ROWS = 64


def _rotl(x, r):
    return (x << np.uint32(r)) | jax.lax.shift_right_logical(x, np.uint32(32 - r))


def _threefry2x32(k0, k1, x0, x1):
    ks2 = k0 ^ k1 ^ np.uint32(0x1BD11BDA)
    x0 = x0 + k0
    x1 = x1 + k1
    rots = ((13, 15, 26, 6), (17, 29, 16, 24))
    sched = ((k1, ks2, 1), (ks2, k0, 2), (k0, k1, 3), (k1, ks2, 4), (ks2, k0, 5))
    for i, (a, b, c) in enumerate(sched):
        for r in rots[i % 2]:
            x0 = x0 + x1
            x1 = _rotl(x1, r)
            x1 = x1 ^ x0
        x0 = x0 + a
        x1 = x1 + b + np.uint32(c)
    return x0, x1


def _erfinv(x):
    w = -jnp.log1p(-x * x)
    ws = w - 2.5
    wb = jnp.sqrt(w) - 3.0
    p_s = jnp.float32(2.81022636e-08)
    for c in (3.43273939e-07, -3.5233877e-06, -4.39150654e-06, 0.00021858087,
              -0.00125372503, -0.00417768164, 0.246640727, 1.50140941):
        p_s = p_s * ws + np.float32(c)
    p_b = jnp.float32(-0.000200214257)
    for c in (0.000100950558, 0.00134934322, -0.00367342844, 0.00573950773,
              -0.0076224613, 0.00943887047, 1.00167406, 2.83297682):
        p_b = p_b * wb + np.float32(c)
    return jnp.where(w < 5.0, p_s, p_b) * x


def _sampler_kernel(key_ref, loc_ref, ls_ref, *refs,
                    rows_total, tile_r, rate, emit_bf16):
    if emit_bf16:
        x_ref, xb_ref, sums_ref, scale_scr = refs
    else:
        x_ref, sums_ref, scale_scr = refs
        xb_ref = None
    r = pl.program_id(0)
    sh = pl.program_id(1)

    @pl.when(sh == 0)
    def _():
        ls = ls_ref[...]
        sp = jnp.maximum(ls, 0.0) + jnp.log1p(jnp.exp(-jnp.abs(ls)))
        scale = jnp.maximum(sp, 1e-5)
        scale_scr[...] = scale
        sums_ref[0, 0, 0] = 0.0
        sums_ref[0, 0, 1] = 0.0
        sums_ref[0, 0, 2] = jnp.sum(jnp.log(scale))
        sums_ref[0, 0, 3] = 0.0

    k0 = key_ref[0]
    k1 = key_ref[1]

    cr = min(_CHUNK_ROWS, tile_r)
    i = jax.lax.broadcasted_iota(jnp.int32, (cr, _LANES), 0)
    j = jax.lax.broadcasted_iota(jnp.int32, (cr, _LANES), 1)
    vec = i * _LANES + j

    def chunk(s_idx, c):
        loc = loc_ref[c * cr:(c + 1) * cr, :]
        scale = scale_scr[c * cr:(c + 1) * cr, :]
        off = (s_idx * rows_total + r * tile_r + c * cr) * _LANES
        cnt = (vec + off).astype(jnp.uint32)
        b0, b1 = _threefry2x32(k0, k1, jnp.zeros_like(cnt), cnt)
        bits = b0 ^ b1
        fb = pltpu.bitcast((bits >> np.uint32(9)) | np.uint32(0x3F800000),
                           jnp.float32) - 1.0
        u = fb * _DIFF + _LO
        eps = _SQRT2 * _erfinv(u)
        y = loc + scale * eps
        xv = jnp.exp(y)
        x_ref[pl.ds(s_idx, 1), c * cr:(c + 1) * cr, :] = xv[None]
        if emit_bf16:
            xb_ref[pl.ds(s_idx, 1), c * cr:(c + 1) * cr, :] = (
                xv.astype(jnp.bfloat16)[None])
        lp = jnp.sum((_CONC - 1.0) * y - rate * xv)
        ent = jnp.sum(0.5 * (eps * eps) + y)
        return lp, ent

    lp_t = 0.0
    ent_t = 0.0
    for s_idx in (sh, sh + _S // 2):
        for c in range(tile_r // cr):
            lp, ent = chunk(s_idx, c)
            lp_t += lp
            ent_t += ent
    sums_ref[0, 0, 0] += lp_t
    sums_ref[0, 0, 1] += ent_t


def _sample_group(kd, loc, log_scale, rate, tile_r, emit_bf16=False):
    n0, n1 = loc.shape
    rows = (n0 * n1) // _LANES
    rt = rows // tile_r
    loc2 = loc.reshape(rows, _LANES)
    ls2 = log_scale.reshape(rows, _LANES)
    out_shape = [jax.ShapeDtypeStruct((_S, rows, _LANES), jnp.float32)]
    out_specs = [pl.BlockSpec((_S, tile_r, _LANES), lambda r, s: (0, r, 0))]
    if emit_bf16:
        out_shape.append(jax.ShapeDtypeStruct((_S, rows, _LANES), jnp.bfloat16))
        out_specs.append(pl.BlockSpec((_S, tile_r, _LANES),
                                      lambda r, s: (0, r, 0)))
    out_shape.append(jax.ShapeDtypeStruct((rt, 1, 4), jnp.float32))
    out_specs.append(pl.BlockSpec((1, 1, 4), lambda r, s: (r, 0, 0),
                                  memory_space=pltpu.MemorySpace.SMEM))
    outs = pl.pallas_call(
        functools.partial(_sampler_kernel, rows_total=rows, tile_r=tile_r,
                          rate=np.float32(rate), emit_bf16=emit_bf16),
        out_shape=tuple(out_shape),
        grid_spec=pltpu.PrefetchScalarGridSpec(
            num_scalar_prefetch=0,
            grid=(rt, _S // 2),
            in_specs=[
                pl.BlockSpec(memory_space=pltpu.MemorySpace.SMEM),
                pl.BlockSpec((tile_r, _LANES), lambda r, s: (r, 0)),
                pl.BlockSpec((tile_r, _LANES), lambda r, s: (r, 0)),
            ],
            out_specs=out_specs,
            scratch_shapes=[pltpu.VMEM((tile_r, _LANES), jnp.float32)]),
        compiler_params=pltpu.CompilerParams(
            dimension_semantics=("arbitrary", "arbitrary"),
            vmem_limit_bytes=48 * 1024 * 1024),
    )(kd, loc2, ls2)
    if emit_bf16:
        x, xb, sums = outs
        return x.reshape(_S, n0, n1), xb.reshape(_S, n0, n1), sums
    x, sums = outs
    return x.reshape(_S, n0, n1), None, sums


def _lgamma_p1(c):
    x = c + 9.0
    p = c + 1.0
    for k in range(2, 9):
        p = p * (c + k)
    lz = jnp.log(x)
    return ((x - 0.5) * lz - x + np.float32(0.5 * _LOG_2PI)
            + 1.0 / (12.0 * x) - 1.0 / (360.0 * x * x * x) - jnp.log(p))


def _ll_kernel(z1_ref, w0_ref, counts_ref, mask_ref, out_ref):
    counts = counts_ref[...]
    keep = 1.0 - mask_ref[...]
    ck = counts * keep
    rate = jnp.dot(z1_ref[0], w0_ref[0], preferred_element_type=jnp.float32)
    prod = rate
    ssum = rate
    for s in range(1, _S):
        rate = jnp.dot(z1_ref[s], w0_ref[s], preferred_element_type=jnp.float32)
        prod = prod * rate
        ssum = ssum + rate
    log_prod = jnp.log(jnp.clip(prod, 1e-30, 1e30))
    out_ref[0, 0, 0] = jnp.sum(ck * log_prod - keep * ssum)
    out_ref[0, 0, 1] = jnp.sum(keep * _lgamma_p1(counts))


def _data_ll_sums(z1_bf16, w0, counts, mask, tile_d):
    dt = _D // tile_d
    return pl.pallas_call(
        _ll_kernel,
        out_shape=jax.ShapeDtypeStruct((dt, 1, 2), jnp.float32),
        grid_spec=pltpu.PrefetchScalarGridSpec(
            num_scalar_prefetch=0,
            grid=(dt,),
            in_specs=[
                pl.BlockSpec((_S, _B, _L0), lambda d: (0, 0, 0)),
                pl.BlockSpec((_S, _L0, tile_d), lambda d: (0, 0, d)),
                pl.BlockSpec((_B, tile_d), lambda d: (0, d)),
                pl.BlockSpec((_B, tile_d), lambda d: (0, d)),
            ],
            out_specs=pl.BlockSpec((1, 1, 2), lambda d: (d, 0, 0),
                                   memory_space=pltpu.MemorySpace.SMEM)),
        compiler_params=pltpu.CompilerParams(
            dimension_semantics=("parallel",),
            vmem_limit_bytes=48 * 1024 * 1024),
    )(z1_bf16, w0, counts, mask)


def kernel(qw1_location, qw1_log_scale, qw0_location, qw0_log_scale,
           qz2_location, qz2_log_scale, qz1_location, qz1_log_scale,
           datapoints_indices, counts, holdout_mask, sample_key):
    params = {
        "qw1": (qw1_location, qw1_log_scale),
        "qw0": (qw0_location, qw0_log_scale),
        "qz2": (qz2_location, qz2_log_scale),
        "qz1": (qz1_location, qz1_log_scale),
    }
    key = jax.random.wrap_key_data(sample_key)
    keys = jax.random.split(key, len(_GROUPS))
    kds = jax.vmap(jax.random.key_data)(keys)

    samples = {}
    samples_bf16 = {}
    lp_sum = 0.0
    ent_sum = 0.0
    logscale_sum = 0.0
    lp_const = 0.0
    total = 0
    for gi, (name, shape, rate, tile_r) in enumerate(_GROUPS):
        loc, ls = params[name]
        x, xb, sums = _sample_group(kds[gi], loc, ls, rate, tile_r,
                                    emit_bf16=name in ("qw0", "qz1"))
        samples[name] = x
        samples_bf16[name] = xb
        lp_sum = lp_sum + jnp.sum(sums[:, 0, 0])
        ent_sum = ent_sum + jnp.sum(sums[:, 0, 1])
        logscale_sum = logscale_sum + jnp.sum(sums[:, 0, 2])
        n = shape[0] * shape[1]
        total += n
        lp_const += n * (_CONC * math.log(rate) - math.lgamma(_CONC))

    log_prior = lp_sum / _S + lp_const
    entropy = ent_sum / _S + logscale_sum + total * (0.5 * _LOG_2PI)

    z1_bf16 = jnp.take(samples_bf16["qz1"], datapoints_indices, axis=1)
    counts_f = counts.astype(jnp.float32)
    parts = _data_ll_sums(z1_bf16, samples_bf16["qw0"], counts_f, holdout_mask,
                          tile_d=512)
    ll_sum = jnp.sum(parts[:, 0, 0])
    lgc_keep = jnp.sum(parts[:, 0, 1])
    data_ll = (_N / _B) * (ll_sum / (_S * _D) - lgc_keep / _D)

    elbo = data_ll + log_prior + entropy
    return elbo, (log_prior, entropy, data_ll), samples


# speedup vs baseline: 2.6971x; 1.2376x over previous
import functools
import math

import numpy as np
import jax
import jax.numpy as jnp
from jax.sharding import Mesh, PartitionSpec as P
from jax.experimental import pallas as pl
from jax.experimental.pallas import tpu as pltpu

_LOG_2PI = math.log(2.0 * math.pi)
_LANES = 128
_CONC = 0.1

_N = 4096
_D = 4096
_L0 = 256
_L1 = 128
_S = 8
_B = 1024

_GROUPS = (
    ("qw1", (_L1, _L0), 0.3, 128),
    ("qw0", (_L0, _D), 0.3, 512),
    ("qz2", (_N, _L1), 0.3, 512),
    ("qz1", (_N, _L0), 2.7, 512),
)

_LO = np.float32(np.nextafter(np.float32(-1.0), np.float32(0.0)))
_DIFF = np.float32(np.float32(1.0) - _LO)
_SQRT2 = np.float32(np.sqrt(2.0))
_CHUNK_ROWS = 64


def _rotl(x, r):
    return (x << np.uint32(r)) | jax.lax.shift_right_logical(x, np.uint32(32 - r))


def _threefry2x32(k0, k1, x0, x1):
    ks2 = k0 ^ k1 ^ np.uint32(0x1BD11BDA)
    x0 = x0 + k0
    x1 = x1 + k1
    rots = ((13, 15, 26, 6), (17, 29, 16, 24))
    sched = ((k1, ks2, 1), (ks2, k0, 2), (k0, k1, 3), (k1, ks2, 4), (ks2, k0, 5))
    for i, (a, b, c) in enumerate(sched):
        for r in rots[i % 2]:
            x0 = x0 + x1
            x1 = _rotl(x1, r)
            x1 = x1 ^ x0
        x0 = x0 + a
        x1 = x1 + b + np.uint32(c)
    return x0, x1


def _erfinv(x):
    w = -jnp.log1p(-x * x)
    ws = w - 2.5
    wb = jnp.sqrt(w) - 3.0
    p_s = jnp.float32(2.81022636e-08)
    for c in (3.43273939e-07, -3.5233877e-06, -4.39150654e-06, 0.00021858087,
              -0.00125372503, -0.00417768164, 0.246640727, 1.50140941):
        p_s = p_s * ws + np.float32(c)
    p_b = jnp.float32(-0.000200214257)
    for c in (0.000100950558, 0.00134934322, -0.00367342844, 0.00573950773,
              -0.0076224613, 0.00943887047, 1.00167406, 2.83297682):
        p_b = p_b * wb + np.float32(c)
    return jnp.where(w < 5.0, p_s, p_b) * x


def _sampler_kernel(key_ref, loc_ref, ls_ref, *refs,
                    rows_total, tile_r, rate, emit_bf16):
    if emit_bf16:
        x_ref, xb_ref, sums_ref, scale_scr = refs
    else:
        x_ref, sums_ref, scale_scr = refs
        xb_ref = None
    r = pl.program_id(0)
    sh = pl.program_id(1)

    @pl.when(sh == 0)
    def _():
        ls = ls_ref[...]
        sp = jnp.maximum(ls, 0.0) + jnp.log1p(jnp.exp(-jnp.abs(ls)))
        scale = jnp.maximum(sp, 1e-5)
        scale_scr[...] = scale
        sums_ref[0, 0, 0] = 0.0
        sums_ref[0, 0, 1] = 0.0
        sums_ref[0, 0, 2] = jnp.sum(jnp.log(scale))
        sums_ref[0, 0, 3] = 0.0

    k0 = key_ref[0]
    k1 = key_ref[1]
    base_row = key_ref[2].astype(jnp.int32)

    cr = min(_CHUNK_ROWS, tile_r)
    i = jax.lax.broadcasted_iota(jnp.int32, (cr, _LANES), 0)
    j = jax.lax.broadcasted_iota(jnp.int32, (cr, _LANES), 1)
    vec = i * _LANES + j

    def chunk(s_idx, c):
        loc = loc_ref[c * cr:(c + 1) * cr, :]
        scale = scale_scr[c * cr:(c + 1) * cr, :]
        off = (s_idx * rows_total + base_row + r * tile_r + c * cr) * _LANES
        cnt = (vec + off).astype(jnp.uint32)
        b0, b1 = _threefry2x32(k0, k1, jnp.zeros_like(cnt), cnt)
        bits = b0 ^ b1
        fb = pltpu.bitcast((bits >> np.uint32(9)) | np.uint32(0x3F800000),
                           jnp.float32) - 1.0
        u = fb * _DIFF + _LO
        eps = _SQRT2 * _erfinv(u)
        y = loc + scale * eps
        xv = jnp.exp(y)
        x_ref[pl.ds(s_idx, 1), c * cr:(c + 1) * cr, :] = xv[None]
        if emit_bf16:
            xb_ref[pl.ds(s_idx, 1), c * cr:(c + 1) * cr, :] = (
                xv.astype(jnp.bfloat16)[None])
        lp = jnp.sum((_CONC - 1.0) * y - rate * xv)
        ent = jnp.sum(0.5 * (eps * eps) + y)
        return lp, ent

    lp_t = 0.0
    ent_t = 0.0
    for s_idx in (sh, sh + _S // 2):
        for c in range(tile_r // cr):
            lp, ent = chunk(s_idx, c)
            lp_t += lp
            ent_t += ent
    sums_ref[0, 0, 0] += lp_t
    sums_ref[0, 0, 1] += ent_t


def _sample_group_local(kd3, loc2, ls2, rate, tile_r, emit_bf16, rows_total):
    rows_local = loc2.shape[0]
    rt = rows_local // tile_r
    out_shape = [jax.ShapeDtypeStruct((_S, rows_local, _LANES), jnp.float32)]
    out_specs = [pl.BlockSpec((_S, tile_r, _LANES), lambda r, s: (0, r, 0))]
    if emit_bf16:
        out_shape.append(jax.ShapeDtypeStruct((_S, rows_local, _LANES),
                                              jnp.bfloat16))
        out_specs.append(pl.BlockSpec((_S, tile_r, _LANES),
                                      lambda r, s: (0, r, 0)))
    out_shape.append(jax.ShapeDtypeStruct((rt, 1, 4), jnp.float32))
    out_specs.append(pl.BlockSpec((1, 1, 4), lambda r, s: (r, 0, 0),
                                  memory_space=pltpu.MemorySpace.SMEM))
    return pl.pallas_call(
        functools.partial(_sampler_kernel, rows_total=rows_total,
                          tile_r=tile_r, rate=np.float32(rate),
                          emit_bf16=emit_bf16),
        out_shape=tuple(out_shape),
        grid_spec=pltpu.PrefetchScalarGridSpec(
            num_scalar_prefetch=0,
            grid=(rt, _S // 2),
            in_specs=[
                pl.BlockSpec(memory_space=pltpu.MemorySpace.SMEM),
                pl.BlockSpec((tile_r, _LANES), lambda r, s: (r, 0)),
                pl.BlockSpec((tile_r, _LANES), lambda r, s: (r, 0)),
            ],
            out_specs=out_specs,
            scratch_shapes=[pltpu.VMEM((tile_r, _LANES), jnp.float32)]),
        compiler_params=pltpu.CompilerParams(
            dimension_semantics=("arbitrary", "arbitrary"),
            vmem_limit_bytes=48 * 1024 * 1024),
    )(kd3, loc2, ls2)


def _sample_group(mesh, kd, loc, log_scale, rate, tile_r, emit_bf16=False):
    n0, n1 = loc.shape
    rows = (n0 * n1) // _LANES
    nc = mesh.shape["c"]
    loc2 = loc.reshape(rows, _LANES)
    ls2 = log_scale.reshape(rows, _LANES)

    def shard_fn(kd_, loc_s, ls_s):
        base = (jax.lax.axis_index("c") * (rows // nc)).astype(jnp.uint32)
        kd3 = jnp.concatenate([kd_, base[None]])
        return _sample_group_local(kd3, loc_s, ls_s, rate, tile_r, emit_bf16,
                                   rows_total=rows)

    if emit_bf16:
        out_specs = (P(None, "c", None), P(None, "c", None), P("c", None, None))
    else:
        out_specs = (P(None, "c", None), P("c", None, None))
    outs = jax.shard_map(shard_fn, mesh=mesh,
                         in_specs=(P(None), P("c", None), P("c", None)),
                         out_specs=out_specs, check_vma=False)(kd, loc2, ls2)
    if emit_bf16:
        x, xb, sums = outs
        return x.reshape(_S, n0, n1), xb.reshape(_S, n0, n1), sums
    x, sums = outs
    return x.reshape(_S, n0, n1), None, sums


def _lgamma_p1(c):
    x = c + 9.0
    p = c + 1.0
    for k in range(2, 9):
        p = p * (c + k)
    lz = jnp.log(x)
    return ((x - 0.5) * lz - x + np.float32(0.5 * _LOG_2PI)
            + 1.0 / (12.0 * x) - 1.0 / (360.0 * x * x * x) - jnp.log(p))


def _ll_kernel(z1_ref, w0_ref, counts_ref, mask_ref, out_ref):
    counts = counts_ref[...]
    keep = 1.0 - mask_ref[...]
    ck = counts * keep
    rate = jnp.dot(z1_ref[0], w0_ref[0], preferred_element_type=jnp.float32)
    prod = rate
    ssum = rate
    for s in range(1, _S):
        rate = jnp.dot(z1_ref[s], w0_ref[s], preferred_element_type=jnp.float32)
        prod = prod * rate
        ssum = ssum + rate
    log_prod = jnp.log(jnp.clip(prod, 1e-30, 1e30))
    out_ref[0, 0, 0] = jnp.sum(ck * log_prod - keep * ssum)
    out_ref[0, 0, 1] = jnp.sum(keep * _lgamma_p1(counts))


def _data_ll_sums(z1_bf16, w0, counts, mask, tile_d):
    s, b, l = z1_bf16.shape
    dt = _D // tile_d
    return pl.pallas_call(
        _ll_kernel,
        out_shape=jax.ShapeDtypeStruct((dt, 1, 2), jnp.float32),
        grid_spec=pltpu.PrefetchScalarGridSpec(
            num_scalar_prefetch=0,
            grid=(dt,),
            in_specs=[
                pl.BlockSpec((s, b, l), lambda d: (0, 0, 0)),
                pl.BlockSpec((s, l, tile_d), lambda d: (0, 0, d)),
                pl.BlockSpec((b, tile_d), lambda d: (0, d)),
                pl.BlockSpec((b, tile_d), lambda d: (0, d)),
            ],
            out_specs=pl.BlockSpec((1, 1, 2), lambda d: (d, 0, 0),
                                   memory_space=pltpu.MemorySpace.SMEM)),
        compiler_params=pltpu.CompilerParams(
            dimension_semantics=("arbitrary",),
            vmem_limit_bytes=48 * 1024 * 1024),
    )(z1_bf16, w0, counts, mask)


def kernel(qw1_location, qw1_log_scale, qw0_location, qw0_log_scale,
           qz2_location, qz2_log_scale, qz1_location, qz1_log_scale,
           datapoints_indices, counts, holdout_mask, sample_key):
    params = {
        "qw1": (qw1_location, qw1_log_scale),
        "qw0": (qw0_location, qw0_log_scale),
        "qz2": (qz2_location, qz2_log_scale),
        "qz1": (qz1_location, qz1_log_scale),
    }
    devs = jax.devices()
    nc = 2 if len(devs) >= 2 else 1
    mesh = Mesh(np.array(devs[:nc]), ("c",))

    key = jax.random.wrap_key_data(sample_key)
    keys = jax.random.split(key, len(_GROUPS))
    kds = jax.vmap(jax.random.key_data)(keys)

    samples = {}
    samples_bf16 = {}
    lp_sum = 0.0
    ent_sum = 0.0
    logscale_sum = 0.0
    lp_const = 0.0
    total = 0
    for gi, (name, shape, rate, tile_r) in enumerate(_GROUPS):
        loc, ls = params[name]
        x, xb, sums = _sample_group(mesh, kds[gi], loc, ls, rate, tile_r,
                                    emit_bf16=name in ("qw0", "qz1"))
        samples[name] = x
        samples_bf16[name] = xb
        lp_sum = lp_sum + jnp.sum(sums[:, 0, 0])
        ent_sum = ent_sum + jnp.sum(sums[:, 0, 1])
        logscale_sum = logscale_sum + jnp.sum(sums[:, 0, 2])
        n = shape[0] * shape[1]
        total += n
        lp_const += n * (_CONC * math.log(rate) - math.lgamma(_CONC))

    log_prior = lp_sum / _S + lp_const
    entropy = ent_sum / _S + logscale_sum + total * (0.5 * _LOG_2PI)

    counts_f = counts.astype(jnp.float32)

    def ll_fn(qz1b, w0b, idxs, cnts, msks):
        z1_sel = jnp.take(qz1b, idxs, axis=1)
        parts = _data_ll_sums(z1_sel, w0b, cnts, msks, tile_d=512)
        return parts[None]

    parts = jax.shard_map(
        ll_fn, mesh=mesh,
        in_specs=(P(None, None, None), P(None, None, None), P("c"),
                  P("c", None), P("c", None)),
        out_specs=P("c", None, None, None), check_vma=False,
    )(samples_bf16["qz1"], samples_bf16["qw0"], datapoints_indices,
      counts_f, holdout_mask)
    ll_sum = jnp.sum(parts[:, :, 0, 0])
    lgc_keep = jnp.sum(parts[:, :, 0, 1])
    data_ll = (_N / _B) * (ll_sum / (_S * _D) - lgc_keep / _D)

    elbo = data_ll + log_prior + entropy
    return elbo, (log_prior, entropy, data_ll), samples
